# baseline (speedup 1.0000x reference)
import functools

import jax
import jax.numpy as jnp
from jax import lax
from jax.experimental import pallas as pl
from jax.experimental.pallas import tpu as pltpu
from jax.experimental.pallas import tpu_sc as plsc

E = 8
D = 768
F = 3072
T = 2048
TM = 128
TPAD = T + E * TM
NT = TPAD // TM
RC = 256

NW = 32
TPW = T // NW


def _router_ln_body(x_ref, wr_ref, br_ref, g_ref, b_ref,
                    xn_ref, dst_ref, gate_ref, own_ref):
    x = x_ref[...]
    logits = jnp.dot(x, wr_ref[...], preferred_element_type=jnp.float32)
    logits = logits + br_ref[...]
    m = jnp.max(logits, axis=-1, keepdims=True)
    ex = jnp.exp(logits - m)
    den = jnp.sum(ex, axis=-1, keepdims=True)
    gate = jnp.max(ex, axis=-1, keepdims=True) / den
    eidx = lax.broadcasted_iota(jnp.int32, (T, E), 1)
    idx = jnp.min(jnp.where(logits >= m, eidx, E), axis=-1, keepdims=True)
    onehot = (eidx == idx).astype(jnp.float32)

    mu = jnp.mean(x, axis=-1, keepdims=True)
    xc = x - mu
    var = jnp.mean(xc * xc, axis=-1, keepdims=True)
    xn_ref[...] = xc / jnp.sqrt(var + 1e-5) * g_ref[...] + b_ref[...]

    ci = lax.broadcasted_iota(jnp.int32, (RC, RC), 0)
    cj = lax.broadcasted_iota(jnp.int32, (RC, RC), 1)
    ltf = (cj < ci).astype(jnp.float32)
    carry = jnp.zeros((1, E), jnp.float32)
    chunks = []
    for c in range(T // RC):
        oh_c = lax.slice(onehot, (c * RC, 0), ((c + 1) * RC, E))
        r_c = jnp.dot(ltf, oh_c, preferred_element_type=jnp.float32,
                      precision=lax.Precision.HIGHEST)
        chunks.append(r_c + carry)
        carry = carry + jnp.sum(oh_c, axis=0, keepdims=True)
    ranks = jnp.concatenate(chunks, axis=0)
    counts = carry

    pc = jnp.ceil(counts / TM) * TM
    ei = lax.broadcasted_iota(jnp.int32, (E, E), 0)
    ej = lax.broadcasted_iota(jnp.int32, (E, E), 1)
    le = (ei <= ej).astype(jnp.float32)
    cum = jnp.dot(pc, le, preferred_element_type=jnp.float32,
                  precision=lax.Precision.HIGHEST)
    off = cum - pc

    rank_t = jnp.sum(ranks * onehot, axis=-1, keepdims=True)
    off_t = jnp.sum(off * onehot, axis=-1, keepdims=True)
    dst_ref[...] = (rank_t + off_t).astype(jnp.int32)
    gate_ref[...] = jnp.broadcast_to(gate, (T, 128))

    ts = lax.broadcasted_iota(jnp.int32, (NT, E), 0).astype(jnp.float32) * TM
    own = jnp.sum((ts >= cum).astype(jnp.int32), axis=-1, keepdims=True)
    own_ref[...] = jnp.minimum(own, E - 1)


def _make_router_ln(interpret=False):
    return pl.pallas_call(
        _router_ln_body,
        out_shape=[
            jax.ShapeDtypeStruct((T, D), jnp.float32),
            jax.ShapeDtypeStruct((T, 1), jnp.int32),
            jax.ShapeDtypeStruct((T, 128), jnp.float32),
            jax.ShapeDtypeStruct((NT, 1), jnp.int32),
        ],
        interpret=interpret,
    )


NF = 4
FB = F // NF


def _gmm_body(own_ref, xs_ref, w1_ref, b1_ref, w2_ref, b2_ref, gs_ref,
              out_ref, acc_ref):
    j = pl.program_id(1)
    h = jnp.dot(xs_ref[...], w1_ref[0], preferred_element_type=jnp.float32)
    h = jnp.maximum(h + b1_ref[0], 0.0)
    y = jnp.dot(h, w2_ref[0], preferred_element_type=jnp.float32)

    @pl.when(j == 0)
    def _():
        acc_ref[...] = y

    @pl.when(j > 0)
    def _():
        acc_ref[...] += y

    @pl.when(j == NF - 1)
    def _():
        out_ref[...] = (acc_ref[...] + b2_ref[0]) * gs_ref[:, :1]


def _make_gmm(interpret=False):
    grid_spec = pltpu.PrefetchScalarGridSpec(
        num_scalar_prefetch=1,
        grid=(NT, NF),
        in_specs=[
            pl.BlockSpec((TM, D), lambda i, j, own: (i, 0)),
            pl.BlockSpec((1, D, FB), lambda i, j, own: (own[i], 0, j)),
            pl.BlockSpec((1, 1, FB), lambda i, j, own: (own[i], 0, j)),
            pl.BlockSpec((1, FB, D), lambda i, j, own: (own[i], j, 0)),
            pl.BlockSpec((1, 1, D), lambda i, j, own: (own[i], 0, 0)),
            pl.BlockSpec((TM, 128), lambda i, j, own: (i, 0)),
        ],
        out_specs=pl.BlockSpec((TM, D), lambda i, j, own: (i, 0)),
        scratch_shapes=[pltpu.VMEM((TM, D), jnp.float32)],
    )
    return pl.pallas_call(
        _gmm_body,
        grid_spec=grid_spec,
        out_shape=jax.ShapeDtypeStruct((TPAD, D), jnp.float32),
        interpret=interpret,
    )


def _make_dispatch():
    mesh = plsc.VectorSubcoreMesh(core_axis_name="c", subcore_axis_name="s")

    @functools.partial(
        pl.kernel, mesh=mesh,
        out_type=[jax.ShapeDtypeStruct((TPAD, D), jnp.float32),
                  jax.ShapeDtypeStruct((TPAD, 128), jnp.float32)],
        scratch_types=[pltpu.VMEM((TPW,), jnp.int32),
                       pltpu.VMEM((TPW, D), jnp.float32),
                       pltpu.VMEM((TPW, 128), jnp.float32),
                       pltpu.SemaphoreType.DMA,
                       pltpu.SemaphoreType.DMA],
    )
    def _dispatch(xn_hbm, dst_hbm, gate_hbm, xs_hbm, gs_hbm,
                  idx_v, rows_v, g_v, sem1, sem2):
        wid = lax.axis_index("s") * 2 + lax.axis_index("c")
        base = wid * TPW
        pltpu.sync_copy(dst_hbm.at[pl.ds(base, TPW)], idx_v)
        pltpu.sync_copy(xn_hbm.at[pl.ds(base, TPW)], rows_v)
        pltpu.sync_copy(gate_hbm.at[pl.ds(base, TPW)], g_v)
        cp1 = pltpu.async_copy(rows_v, xs_hbm.at[idx_v], sem1)
        cp2 = pltpu.async_copy(g_v, gs_hbm.at[idx_v], sem2)
        cp1.wait()
        cp2.wait()

    return _dispatch


def _make_collect():
    mesh = plsc.VectorSubcoreMesh(core_axis_name="c", subcore_axis_name="s")

    @functools.partial(
        pl.kernel, mesh=mesh,
        out_type=jax.ShapeDtypeStruct((T, D), jnp.float32),
        scratch_types=[pltpu.VMEM((TPW,), jnp.int32),
                       pltpu.VMEM((TPW, D), jnp.float32),
                       pltpu.SemaphoreType.DMA],
    )
    def _collect(ys_hbm, dst_hbm, y_hbm, idx_v, rows_v, sem):
        wid = lax.axis_index("s") * 2 + lax.axis_index("c")
        base = wid * TPW
        pltpu.sync_copy(dst_hbm.at[pl.ds(base, TPW)], idx_v)
        pltpu.async_copy(ys_hbm.at[idx_v], rows_v, sem).wait()
        pltpu.sync_copy(rows_v, y_hbm.at[pl.ds(base, TPW)])

    return _collect


def kernel(x, Wr, br, gamma, beta, W1, b1, W2, b2):
    xf = x.reshape(T, D)
    xn, dst, gate16, owner = _make_router_ln()(
        xf, Wr, br.reshape(1, E), gamma.reshape(1, D), beta.reshape(1, D))
    dst1 = dst.reshape(T)
    xs, gs = _make_dispatch()(xn, dst1, gate16)
    ys = _make_gmm()(owner.reshape(NT), xs, W1, b1.reshape(E, 1, F),
                     W2, b2.reshape(E, 1, D), gs)
    y = _make_collect()(ys, dst1)
    return y.reshape(1, T, D)


# speedup vs baseline: 1.6072x; 1.6072x over previous
import functools

import jax
import jax.numpy as jnp
from jax import lax
from jax.experimental import pallas as pl
from jax.experimental.pallas import tpu as pltpu
from jax.experimental.pallas import tpu_sc as plsc

E = 8
D = 768
F = 3072
T = 2048
TM = 128
TPAD = T + E * TM
NT = TPAD // TM
RC = 256

NW = 32
TPW = T // NW


def _router_body(x_ref, wr_ref, br_ref, dst_ref, gate_ref, own_ref):
    x = x_ref[...]
    logits = jnp.dot(x, wr_ref[...], preferred_element_type=jnp.float32)
    logits = logits + br_ref[...]
    m = jnp.max(logits, axis=-1, keepdims=True)
    ex = jnp.exp(logits - m)
    den = jnp.sum(ex, axis=-1, keepdims=True)
    gate = jnp.max(ex, axis=-1, keepdims=True) / den
    eidx = lax.broadcasted_iota(jnp.int32, (T, E), 1)
    idx = jnp.min(jnp.where(logits >= m, eidx, E), axis=-1, keepdims=True)
    onehot = (eidx == idx).astype(jnp.float32)

    ci = lax.broadcasted_iota(jnp.int32, (RC, RC), 0)
    cj = lax.broadcasted_iota(jnp.int32, (RC, RC), 1)
    ltf = (cj < ci).astype(jnp.float32)
    carry = jnp.zeros((1, E), jnp.float32)
    chunks = []
    for c in range(T // RC):
        oh_c = lax.slice(onehot, (c * RC, 0), ((c + 1) * RC, E))
        r_c = jnp.dot(ltf, oh_c, preferred_element_type=jnp.float32,
                      precision=lax.Precision.HIGHEST)
        chunks.append(r_c + carry)
        carry = carry + jnp.sum(oh_c, axis=0, keepdims=True)
    ranks = jnp.concatenate(chunks, axis=0)
    counts = carry

    pc = jnp.ceil(counts / TM) * TM
    ei = lax.broadcasted_iota(jnp.int32, (E, E), 0)
    ej = lax.broadcasted_iota(jnp.int32, (E, E), 1)
    le = (ei <= ej).astype(jnp.float32)
    cum = jnp.dot(pc, le, preferred_element_type=jnp.float32,
                  precision=lax.Precision.HIGHEST)
    off = cum - pc

    rank_t = jnp.sum(ranks * onehot, axis=-1, keepdims=True)
    off_t = jnp.sum(off * onehot, axis=-1, keepdims=True)
    dst_ref[...] = (rank_t + off_t).astype(jnp.int32)
    gate_ref[...] = jnp.broadcast_to(gate, (T, 128))

    ts = lax.broadcasted_iota(jnp.int32, (NT, E), 0).astype(jnp.float32) * TM
    own = jnp.sum((ts >= cum).astype(jnp.int32), axis=-1, keepdims=True)
    own_ref[...] = jnp.minimum(own, E - 1)


def _make_router(interpret=False):
    return pl.pallas_call(
        _router_body,
        out_shape=[
            jax.ShapeDtypeStruct((T, 1), jnp.int32),
            jax.ShapeDtypeStruct((T, 128), jnp.float32),
            jax.ShapeDtypeStruct((NT, 1), jnp.int32),
        ],
        interpret=interpret,
    )


def _gmm_body(own_ref, xs_ref, w1_ref, b1_ref, w2_ref, b2_ref, gs_ref,
              gamma_ref, beta_ref, out_ref):
    x = xs_ref[...]
    mu = jnp.mean(x, axis=-1, keepdims=True)
    xc = x - mu
    var = jnp.mean(xc * xc, axis=-1, keepdims=True)
    xn = xc / jnp.sqrt(var + 1e-5) * gamma_ref[...] + beta_ref[...]
    h = jnp.dot(xn, w1_ref[0], preferred_element_type=jnp.float32)
    h = jnp.maximum(h + b1_ref[0], 0.0)
    y = jnp.dot(h, w2_ref[0], preferred_element_type=jnp.float32)
    out_ref[...] = (y + b2_ref[0]) * gs_ref[:, :1]


def _make_gmm(interpret=False):
    grid_spec = pltpu.PrefetchScalarGridSpec(
        num_scalar_prefetch=1,
        grid=(NT,),
        in_specs=[
            pl.BlockSpec((TM, D), lambda i, own: (i, 0)),
            pl.BlockSpec((1, D, F), lambda i, own: (own[i], 0, 0)),
            pl.BlockSpec((1, 1, F), lambda i, own: (own[i], 0, 0)),
            pl.BlockSpec((1, F, D), lambda i, own: (own[i], 0, 0)),
            pl.BlockSpec((1, 1, D), lambda i, own: (own[i], 0, 0)),
            pl.BlockSpec((TM, 128), lambda i, own: (i, 0)),
            pl.BlockSpec((1, D), lambda i, own: (0, 0)),
            pl.BlockSpec((1, D), lambda i, own: (0, 0)),
        ],
        out_specs=pl.BlockSpec((TM, D), lambda i, own: (i, 0)),
    )
    return pl.pallas_call(
        _gmm_body,
        grid_spec=grid_spec,
        out_shape=jax.ShapeDtypeStruct((TPAD, D), jnp.float32),
        interpret=interpret,
    )


def _make_dispatch():
    mesh = plsc.VectorSubcoreMesh(core_axis_name="c", subcore_axis_name="s")

    @functools.partial(
        pl.kernel, mesh=mesh,
        out_type=[jax.ShapeDtypeStruct((TPAD, D), jnp.float32),
                  jax.ShapeDtypeStruct((TPAD, 128), jnp.float32)],
        scratch_types=[pltpu.VMEM((TPW,), jnp.int32),
                       pltpu.VMEM((TPW, D), jnp.float32),
                       pltpu.VMEM((TPW, 128), jnp.float32),
                       pltpu.SemaphoreType.DMA,
                       pltpu.SemaphoreType.DMA],
    )
    def _dispatch(xn_hbm, dst_hbm, gate_hbm, xs_hbm, gs_hbm,
                  idx_v, rows_v, g_v, sem1, sem2):
        wid = lax.axis_index("s") * 2 + lax.axis_index("c")
        base = wid * TPW
        pltpu.sync_copy(dst_hbm.at[pl.ds(base, TPW)], idx_v)
        pltpu.sync_copy(xn_hbm.at[pl.ds(base, TPW)], rows_v)
        pltpu.sync_copy(gate_hbm.at[pl.ds(base, TPW)], g_v)
        cp1 = pltpu.async_copy(rows_v, xs_hbm.at[idx_v], sem1)
        cp2 = pltpu.async_copy(g_v, gs_hbm.at[idx_v], sem2)
        cp1.wait()
        cp2.wait()

    return _dispatch


def _make_collect():
    mesh = plsc.VectorSubcoreMesh(core_axis_name="c", subcore_axis_name="s")

    @functools.partial(
        pl.kernel, mesh=mesh,
        out_type=jax.ShapeDtypeStruct((T, D), jnp.float32),
        scratch_types=[pltpu.VMEM((TPW,), jnp.int32),
                       pltpu.VMEM((TPW, D), jnp.float32),
                       pltpu.SemaphoreType.DMA],
    )
    def _collect(ys_hbm, dst_hbm, y_hbm, idx_v, rows_v, sem):
        wid = lax.axis_index("s") * 2 + lax.axis_index("c")
        base = wid * TPW
        pltpu.sync_copy(dst_hbm.at[pl.ds(base, TPW)], idx_v)
        pltpu.async_copy(ys_hbm.at[idx_v], rows_v, sem).wait()
        pltpu.sync_copy(rows_v, y_hbm.at[pl.ds(base, TPW)])

    return _collect


def kernel(x, Wr, br, gamma, beta, W1, b1, W2, b2):
    xf = x.reshape(T, D)
    dst, gate128, owner = _make_router()(xf, Wr, br.reshape(1, E))
    dst1 = dst.reshape(T)
    xs, gs = _make_dispatch()(xf, dst1, gate128)
    ys = _make_gmm()(owner.reshape(NT), xs, W1, b1.reshape(E, 1, F),
                     W2, b2.reshape(E, 1, D), gs,
                     gamma.reshape(1, D), beta.reshape(1, D))
    y = _make_collect()(ys, dst1)
    return y.reshape(1, T, D)


# speedup vs baseline: 2.0253x; 1.2602x over previous
import functools

import jax
import jax.numpy as jnp
from jax import lax
from jax.experimental import pallas as pl
from jax.experimental.pallas import tpu as pltpu
from jax.experimental.pallas import tpu_sc as plsc

E = 8
D = 768
F = 3072
T = 2048
TM = 128
TPAD = T + E * TM
NT = TPAD // TM
RC = 256

NW = 32
TPW = T // NW


def _router_body(x_ref, wr_ref, br_ref, dst_ref, gate_ref,
                 ea_ref, offs_ref, nt_ref):
    x = x_ref[...]
    logits = jnp.dot(x, wr_ref[...], preferred_element_type=jnp.float32)
    logits = logits + br_ref[...]
    m = jnp.max(logits, axis=-1, keepdims=True)
    ex = jnp.exp(logits - m)
    den = jnp.sum(ex, axis=-1, keepdims=True)
    gate = jnp.max(ex, axis=-1, keepdims=True) / den
    eidx = lax.broadcasted_iota(jnp.int32, (T, E), 1)
    idx = jnp.min(jnp.where(logits >= m, eidx, E), axis=-1, keepdims=True)
    onehot = (eidx == idx).astype(jnp.float32)

    ci = lax.broadcasted_iota(jnp.int32, (RC, RC), 0)
    cj = lax.broadcasted_iota(jnp.int32, (RC, RC), 1)
    ltf = (cj < ci).astype(jnp.float32)
    carry = jnp.zeros((1, E), jnp.float32)
    chunks = []
    for c in range(T // RC):
        oh_c = lax.slice(onehot, (c * RC, 0), ((c + 1) * RC, E))
        r_c = jnp.dot(ltf, oh_c, preferred_element_type=jnp.float32,
                      precision=lax.Precision.HIGHEST)
        chunks.append(r_c + carry)
        carry = carry + jnp.sum(oh_c, axis=0, keepdims=True)
    ranks = jnp.concatenate(chunks, axis=0)
    counts = carry

    pc = jnp.ceil(counts / TM) * TM
    ei = lax.broadcasted_iota(jnp.int32, (E, E), 0)
    ej = lax.broadcasted_iota(jnp.int32, (E, E), 1)
    le = (ei <= ej).astype(jnp.float32)
    cum = jnp.dot(pc, le, preferred_element_type=jnp.float32,
                  precision=lax.Precision.HIGHEST)
    off = cum - pc

    rank_t = jnp.sum(ranks * onehot, axis=-1, keepdims=True)
    off_t = jnp.sum(off * onehot, axis=-1, keepdims=True)
    dst_ref[...] = (rank_t + off_t).astype(jnp.int32)
    gate_ref[...] = jnp.broadcast_to(gate, (T, 128))

    present = (counts > 0.0).astype(jnp.float32)
    lt_strict = (ei < ej).astype(jnp.float32)
    rankp = jnp.dot(present, lt_strict,
                    preferred_element_type=jnp.float32,
                    precision=lax.Precision.HIGHEST)
    np_ = jnp.sum(present)
    srow = lax.broadcasted_iota(jnp.int32, (E, E), 0).astype(jnp.float32)
    ecol = lax.broadcasted_iota(jnp.int32, (E, E), 1).astype(jnp.float32)
    mse = (jnp.broadcast_to(rankp, (E, E)) == srow) * \
        jnp.broadcast_to(present, (E, E))
    ea_col = jnp.sum(ecol * mse, axis=-1, keepdims=True)
    is_last = present * (rankp == (np_ - 1.0)).astype(jnp.float32)
    lastp = jnp.sum(is_last * ecol[:1, :])
    beyond = (lax.broadcasted_iota(jnp.int32, (E, 1), 0).astype(jnp.float32)
              >= np_).astype(jnp.float32)
    ea_ref[...] = (ea_col + beyond * lastp).astype(jnp.int32)
    offs_ref[...] = jnp.sum(jnp.broadcast_to(off, (E, E)) * mse,
                            axis=-1, keepdims=True).astype(jnp.int32)
    nt_ref[...] = jnp.sum(jnp.broadcast_to(pc / TM, (E, E)) * mse,
                          axis=-1, keepdims=True).astype(jnp.int32)


def _make_router(interpret=False):
    return pl.pallas_call(
        _router_body,
        out_shape=[
            jax.ShapeDtypeStruct((T, 1), jnp.int32),
            jax.ShapeDtypeStruct((T, 128), jnp.float32),
            jax.ShapeDtypeStruct((E, 1), jnp.int32),
            jax.ShapeDtypeStruct((E, 1), jnp.int32),
            jax.ShapeDtypeStruct((E, 1), jnp.int32),
        ],
        interpret=interpret,
    )


def _gmm_body(ea_ref, offs_ref, nt_ref, xs_ref, w1_ref, b1_ref, w2_ref,
              b2_ref, gs_ref, gamma_ref, beta_ref, out_ref):
    s = pl.program_id(0)
    n = nt_ref[s]
    base = pl.multiple_of(offs_ref[s], TM)

    def tile(k, carry):
        r0 = pl.multiple_of(base + k * TM, TM)
        x = xs_ref[pl.ds(r0, TM), :]
        mu = jnp.mean(x, axis=-1, keepdims=True)
        xc = x - mu
        var = jnp.mean(xc * xc, axis=-1, keepdims=True)
        xn = xc / jnp.sqrt(var + 1e-5) * gamma_ref[...] + beta_ref[...]
        h = jnp.dot(xn, w1_ref[0], preferred_element_type=jnp.float32)
        h = jnp.maximum(h + b1_ref[0], 0.0)
        y = jnp.dot(h, w2_ref[0], preferred_element_type=jnp.float32)
        out_ref[pl.ds(r0, TM), :] = \
            (y + b2_ref[0]) * gs_ref[pl.ds(r0, TM), :1]
        return carry

    lax.fori_loop(0, n, tile, 0)


def _make_gmm(interpret=False):
    grid_spec = pltpu.PrefetchScalarGridSpec(
        num_scalar_prefetch=3,
        grid=(E,),
        in_specs=[
            pl.BlockSpec((TPAD, D), lambda s, ea, of, nt: (0, 0)),
            pl.BlockSpec((1, D, F), lambda s, ea, of, nt: (ea[s], 0, 0)),
            pl.BlockSpec((1, 1, F), lambda s, ea, of, nt: (ea[s], 0, 0)),
            pl.BlockSpec((1, F, D), lambda s, ea, of, nt: (ea[s], 0, 0)),
            pl.BlockSpec((1, 1, D), lambda s, ea, of, nt: (ea[s], 0, 0)),
            pl.BlockSpec((TPAD, 128), lambda s, ea, of, nt: (0, 0)),
            pl.BlockSpec((1, D), lambda s, ea, of, nt: (0, 0)),
            pl.BlockSpec((1, D), lambda s, ea, of, nt: (0, 0)),
        ],
        out_specs=pl.BlockSpec((TPAD, D), lambda s, ea, of, nt: (0, 0)),
    )
    return pl.pallas_call(
        _gmm_body,
        grid_spec=grid_spec,
        out_shape=jax.ShapeDtypeStruct((TPAD, D), jnp.float32),
        compiler_params=pltpu.CompilerParams(
            vmem_limit_bytes=100 * 1024 * 1024),
        interpret=interpret,
    )


def _make_dispatch():
    mesh = plsc.VectorSubcoreMesh(core_axis_name="c", subcore_axis_name="s")

    @functools.partial(
        pl.kernel, mesh=mesh,
        out_type=[jax.ShapeDtypeStruct((TPAD, D), jnp.float32),
                  jax.ShapeDtypeStruct((TPAD, 128), jnp.float32)],
        scratch_types=[pltpu.VMEM((TPW,), jnp.int32),
                       pltpu.VMEM((TPW, D), jnp.float32),
                       pltpu.VMEM((TPW, 128), jnp.float32),
                       pltpu.SemaphoreType.DMA,
                       pltpu.SemaphoreType.DMA],
    )
    def _dispatch(xn_hbm, dst_hbm, gate_hbm, xs_hbm, gs_hbm,
                  idx_v, rows_v, g_v, sem1, sem2):
        wid = lax.axis_index("s") * 2 + lax.axis_index("c")
        base = wid * TPW
        pltpu.sync_copy(dst_hbm.at[pl.ds(base, TPW)], idx_v)
        pltpu.sync_copy(xn_hbm.at[pl.ds(base, TPW)], rows_v)
        pltpu.sync_copy(gate_hbm.at[pl.ds(base, TPW)], g_v)
        cp1 = pltpu.async_copy(rows_v, xs_hbm.at[idx_v], sem1)
        cp2 = pltpu.async_copy(g_v, gs_hbm.at[idx_v], sem2)
        cp1.wait()
        cp2.wait()

    return _dispatch


def _make_collect():
    mesh = plsc.VectorSubcoreMesh(core_axis_name="c", subcore_axis_name="s")

    @functools.partial(
        pl.kernel, mesh=mesh,
        out_type=jax.ShapeDtypeStruct((T, D), jnp.float32),
        scratch_types=[pltpu.VMEM((TPW,), jnp.int32),
                       pltpu.VMEM((TPW, D), jnp.float32),
                       pltpu.SemaphoreType.DMA],
    )
    def _collect(ys_hbm, dst_hbm, y_hbm, idx_v, rows_v, sem):
        wid = lax.axis_index("s") * 2 + lax.axis_index("c")
        base = wid * TPW
        pltpu.sync_copy(dst_hbm.at[pl.ds(base, TPW)], idx_v)
        pltpu.async_copy(ys_hbm.at[idx_v], rows_v, sem).wait()
        pltpu.sync_copy(rows_v, y_hbm.at[pl.ds(base, TPW)])

    return _collect


def kernel(x, Wr, br, gamma, beta, W1, b1, W2, b2):
    xf = x.reshape(T, D)
    dst, gate128, ea, offs, nt = _make_router()(xf, Wr, br.reshape(1, E))
    dst1 = dst.reshape(T)
    xs, gs = _make_dispatch()(xf, dst1, gate128)
    ys = _make_gmm()(ea.reshape(E), offs.reshape(E), nt.reshape(E),
                     xs, W1, b1.reshape(E, 1, F),
                     W2, b2.reshape(E, 1, D), gs,
                     gamma.reshape(1, D), beta.reshape(1, D))
    y = _make_collect()(ys, dst1)
    return y.reshape(1, T, D)


# speedup vs baseline: 2.0466x; 1.0105x over previous
import functools

import jax
import jax.numpy as jnp
from jax import lax
from jax.experimental import pallas as pl
from jax.experimental.pallas import tpu as pltpu
from jax.experimental.pallas import tpu_sc as plsc

E = 8
D = 768
F = 3072
T = 2048
TM = 128
TPAD = T + E * TM
NT = TPAD // TM
RC = 256

NW = 32
TPW = T // NW


def _router_body(x_ref, wr_ref, br_ref, dst_ref, gate_ref,
                 ea_ref, offs_ref, nt_ref):
    x = x_ref[...]
    logits = jnp.dot(x, wr_ref[...], preferred_element_type=jnp.float32)
    logits = logits + br_ref[...]
    m = jnp.max(logits, axis=-1, keepdims=True)
    ex = jnp.exp(logits - m)
    den = jnp.sum(ex, axis=-1, keepdims=True)
    gate = jnp.max(ex, axis=-1, keepdims=True) / den
    eidx = lax.broadcasted_iota(jnp.int32, (T, E), 1)
    idx = jnp.min(jnp.where(logits >= m, eidx, E), axis=-1, keepdims=True)
    onehot = (eidx == idx).astype(jnp.float32)

    ci = lax.broadcasted_iota(jnp.int32, (RC, RC), 0)
    cj = lax.broadcasted_iota(jnp.int32, (RC, RC), 1)
    ltf = (cj < ci).astype(jnp.float32)
    carry = jnp.zeros((1, E), jnp.float32)
    chunks = []
    for c in range(T // RC):
        oh_c = lax.slice(onehot, (c * RC, 0), ((c + 1) * RC, E))
        r_c = jnp.dot(ltf, oh_c, preferred_element_type=jnp.float32,
                      precision=lax.Precision.HIGHEST)
        chunks.append(r_c + carry)
        carry = carry + jnp.sum(oh_c, axis=0, keepdims=True)
    ranks = jnp.concatenate(chunks, axis=0)
    counts = carry

    pc = jnp.ceil(counts / TM) * TM
    ei = lax.broadcasted_iota(jnp.int32, (E, E), 0)
    ej = lax.broadcasted_iota(jnp.int32, (E, E), 1)
    le = (ei <= ej).astype(jnp.float32)
    cum = jnp.dot(pc, le, preferred_element_type=jnp.float32,
                  precision=lax.Precision.HIGHEST)
    off = cum - pc

    rank_t = jnp.sum(ranks * onehot, axis=-1, keepdims=True)
    off_t = jnp.sum(off * onehot, axis=-1, keepdims=True)
    dst_ref[...] = (rank_t + off_t).astype(jnp.int32)
    gate_ref[...] = jnp.broadcast_to(gate, (T, 128))

    present = (counts > 0.0).astype(jnp.float32)
    lt_strict = (ei < ej).astype(jnp.float32)
    rankp = jnp.dot(present, lt_strict,
                    preferred_element_type=jnp.float32,
                    precision=lax.Precision.HIGHEST)
    np_ = jnp.sum(present)
    srow = lax.broadcasted_iota(jnp.int32, (E, E), 0).astype(jnp.float32)
    ecol = lax.broadcasted_iota(jnp.int32, (E, E), 1).astype(jnp.float32)
    mse = (jnp.broadcast_to(rankp, (E, E)) == srow) * \
        jnp.broadcast_to(present, (E, E))
    ea_col = jnp.sum(ecol * mse, axis=-1, keepdims=True)
    is_last = present * (rankp == (np_ - 1.0)).astype(jnp.float32)
    lastp = jnp.sum(is_last * ecol[:1, :])
    beyond = (lax.broadcasted_iota(jnp.int32, (E, 1), 0).astype(jnp.float32)
              >= np_).astype(jnp.float32)
    ea_ref[...] = (ea_col + beyond * lastp).astype(jnp.int32)
    offs_ref[...] = jnp.sum(jnp.broadcast_to(off, (E, E)) * mse,
                            axis=-1, keepdims=True).astype(jnp.int32)
    nt_ref[...] = jnp.sum(jnp.broadcast_to(pc / TM, (E, E)) * mse,
                          axis=-1, keepdims=True).astype(jnp.int32)


def _make_router(interpret=False):
    return pl.pallas_call(
        _router_body,
        out_shape=[
            jax.ShapeDtypeStruct((T, 1), jnp.int32),
            jax.ShapeDtypeStruct((T, 128), jnp.float32),
            jax.ShapeDtypeStruct((E, 1), jnp.int32),
            jax.ShapeDtypeStruct((E, 1), jnp.int32),
            jax.ShapeDtypeStruct((E, 1), jnp.int32),
        ],
        interpret=interpret,
    )


def _gmm_body(ea_ref, offs_ref, nt_ref, xs_ref, w1_ref, b1_ref, w2_ref,
              b2_ref, gs_ref, gamma_ref, beta_ref, out_ref):
    s = pl.program_id(0)
    n = nt_ref[s]
    base = pl.multiple_of(offs_ref[s], TM)

    def tile(k, carry):
        r0 = pl.multiple_of(base + k * TM, TM)
        x = xs_ref[pl.ds(r0, TM), :]
        mu = jnp.mean(x, axis=-1, keepdims=True)
        xc = x - mu
        var = jnp.mean(xc * xc, axis=-1, keepdims=True)
        xn = xc / jnp.sqrt(var + 1e-5) * gamma_ref[...] + beta_ref[...]
        h = jnp.dot(xn, w1_ref[0], preferred_element_type=jnp.float32)
        h = jnp.maximum(h + b1_ref[0], 0.0)
        y = jnp.dot(h, w2_ref[0], preferred_element_type=jnp.float32)
        out_ref[pl.ds(r0, TM), :] = \
            (y + b2_ref[0]) * gs_ref[pl.ds(r0, TM), :1]
        return carry

    lax.fori_loop(0, n, tile, 0)


def _make_gmm(interpret=False):
    grid_spec = pltpu.PrefetchScalarGridSpec(
        num_scalar_prefetch=3,
        grid=(E,),
        in_specs=[
            pl.BlockSpec((TPAD, D), lambda s, ea, of, nt: (0, 0)),
            pl.BlockSpec((1, D, F), lambda s, ea, of, nt: (ea[s], 0, 0)),
            pl.BlockSpec((1, 1, F), lambda s, ea, of, nt: (ea[s], 0, 0)),
            pl.BlockSpec((1, F, D), lambda s, ea, of, nt: (ea[s], 0, 0)),
            pl.BlockSpec((1, 1, D), lambda s, ea, of, nt: (ea[s], 0, 0)),
            pl.BlockSpec((TPAD, 128), lambda s, ea, of, nt: (0, 0)),
            pl.BlockSpec((1, D), lambda s, ea, of, nt: (0, 0)),
            pl.BlockSpec((1, D), lambda s, ea, of, nt: (0, 0)),
        ],
        out_specs=pl.BlockSpec((TPAD, D), lambda s, ea, of, nt: (0, 0)),
    )
    return pl.pallas_call(
        _gmm_body,
        grid_spec=grid_spec,
        out_shape=jax.ShapeDtypeStruct((TPAD, D), jnp.float32),
        compiler_params=pltpu.CompilerParams(
            vmem_limit_bytes=100 * 1024 * 1024),
        interpret=interpret,
    )


def _make_dispatch():
    mesh = plsc.VectorSubcoreMesh(core_axis_name="c", subcore_axis_name="s")

    @functools.partial(
        pl.kernel, mesh=mesh,
        out_type=[jax.ShapeDtypeStruct((TPAD, D), jnp.float32),
                  jax.ShapeDtypeStruct((TPAD, 128), jnp.float32)],
        scratch_types=[pltpu.VMEM((TPW,), jnp.int32),
                       pltpu.VMEM((TPW, D), jnp.float32),
                       pltpu.VMEM((TPW, 128), jnp.float32),
                       pltpu.SemaphoreType.DMA,
                       pltpu.SemaphoreType.DMA,
                       pltpu.SemaphoreType.DMA],
    )
    def _dispatch(xn_hbm, dst_hbm, gate_hbm, xs_hbm, gs_hbm,
                  idx_v, rows_v, g_v, sem1, sem2, sem3):
        wid = lax.axis_index("s") * 2 + lax.axis_index("c")
        base = wid * TPW
        ld1 = pltpu.async_copy(dst_hbm.at[pl.ds(base, TPW)], idx_v, sem1)
        ld2 = pltpu.async_copy(xn_hbm.at[pl.ds(base, TPW)], rows_v, sem2)
        ld3 = pltpu.async_copy(gate_hbm.at[pl.ds(base, TPW)], g_v, sem3)
        ld1.wait()
        ld2.wait()
        ld3.wait()
        cp1 = pltpu.async_copy(rows_v, xs_hbm.at[idx_v], sem1)
        cp2 = pltpu.async_copy(g_v, gs_hbm.at[idx_v], sem2)
        cp1.wait()
        cp2.wait()

    return _dispatch


def _make_collect():
    mesh = plsc.VectorSubcoreMesh(core_axis_name="c", subcore_axis_name="s")

    @functools.partial(
        pl.kernel, mesh=mesh,
        out_type=jax.ShapeDtypeStruct((T, D), jnp.float32),
        scratch_types=[pltpu.VMEM((TPW,), jnp.int32),
                       pltpu.VMEM((TPW, D), jnp.float32),
                       pltpu.SemaphoreType.DMA],
    )
    def _collect(ys_hbm, dst_hbm, y_hbm, idx_v, rows_v, sem):
        wid = lax.axis_index("s") * 2 + lax.axis_index("c")
        base = wid * TPW
        pltpu.sync_copy(dst_hbm.at[pl.ds(base, TPW)], idx_v)
        pltpu.async_copy(ys_hbm.at[idx_v], rows_v, sem).wait()
        pltpu.sync_copy(rows_v, y_hbm.at[pl.ds(base, TPW)])

    return _collect


def kernel(x, Wr, br, gamma, beta, W1, b1, W2, b2):
    xf = x.reshape(T, D)
    dst, gate128, ea, offs, nt = _make_router()(xf, Wr, br.reshape(1, E))
    dst1 = dst.reshape(T)
    xs, gs = _make_dispatch()(xf, dst1, gate128)
    ys = _make_gmm()(ea.reshape(E), offs.reshape(E), nt.reshape(E),
                     xs, W1, b1.reshape(E, 1, F),
                     W2, b2.reshape(E, 1, D), gs,
                     gamma.reshape(1, D), beta.reshape(1, D))
    y = _make_collect()(ys, dst1)
    return y.reshape(1, T, D)


# speedup vs baseline: 2.0745x; 1.0136x over previous
import functools

import jax
import jax.numpy as jnp
from jax import lax
from jax.experimental import pallas as pl
from jax.experimental.pallas import tpu as pltpu
from jax.experimental.pallas import tpu_sc as plsc

E = 8
D = 768
F = 3072
T = 2048
TM = 128
TPAD = T + E * TM
NT = TPAD // TM
RC = 256

NW = 32
TPW = T // NW


def _router_body(x_ref, wr_ref, br_ref, dst_ref, gate_ref,
                 ea_ref, offs_ref, nt_ref):
    x = x_ref[...]
    logits = jnp.dot(x, wr_ref[...], preferred_element_type=jnp.float32)
    logits = logits + br_ref[...]
    m = jnp.max(logits, axis=-1, keepdims=True)
    ex = jnp.exp(logits - m)
    den = jnp.sum(ex, axis=-1, keepdims=True)
    gate = jnp.max(ex, axis=-1, keepdims=True) / den
    eidx = lax.broadcasted_iota(jnp.int32, (T, E), 1)
    idx = jnp.min(jnp.where(logits >= m, eidx, E), axis=-1, keepdims=True)
    onehot = (eidx == idx).astype(jnp.float32)

    ci = lax.broadcasted_iota(jnp.int32, (RC, RC), 0)
    cj = lax.broadcasted_iota(jnp.int32, (RC, RC), 1)
    ltf = (cj < ci).astype(jnp.float32)
    carry = jnp.zeros((1, E), jnp.float32)
    chunks = []
    for c in range(T // RC):
        oh_c = lax.slice(onehot, (c * RC, 0), ((c + 1) * RC, E))
        r_c = jnp.dot(ltf, oh_c, preferred_element_type=jnp.float32)
        chunks.append(r_c + carry)
        carry = carry + jnp.sum(oh_c, axis=0, keepdims=True)
    ranks = jnp.concatenate(chunks, axis=0)
    counts = carry

    pc = jnp.ceil(counts / TM) * TM
    ei = lax.broadcasted_iota(jnp.int32, (E, E), 0)
    ej = lax.broadcasted_iota(jnp.int32, (E, E), 1)
    le = (ei <= ej).astype(jnp.float32)
    cum = jnp.dot(pc, le, preferred_element_type=jnp.float32)
    off = cum - pc

    rank_t = jnp.sum(ranks * onehot, axis=-1, keepdims=True)
    off_t = jnp.sum(off * onehot, axis=-1, keepdims=True)
    dst_ref[...] = (rank_t + off_t).astype(jnp.int32)
    gate_ref[...] = jnp.broadcast_to(gate, (T, 128))

    present = (counts > 0.0).astype(jnp.float32)
    lt_strict = (ei < ej).astype(jnp.float32)
    rankp = jnp.dot(present, lt_strict,
                    preferred_element_type=jnp.float32)
    np_ = jnp.sum(present)
    srow = lax.broadcasted_iota(jnp.int32, (E, E), 0).astype(jnp.float32)
    ecol = lax.broadcasted_iota(jnp.int32, (E, E), 1).astype(jnp.float32)
    mse = (jnp.broadcast_to(rankp, (E, E)) == srow) * \
        jnp.broadcast_to(present, (E, E))
    ea_col = jnp.sum(ecol * mse, axis=-1, keepdims=True)
    is_last = present * (rankp == (np_ - 1.0)).astype(jnp.float32)
    lastp = jnp.sum(is_last * ecol[:1, :])
    beyond = (lax.broadcasted_iota(jnp.int32, (E, 1), 0).astype(jnp.float32)
              >= np_).astype(jnp.float32)
    ea_ref[...] = (ea_col + beyond * lastp).astype(jnp.int32)
    offs_ref[...] = jnp.sum(jnp.broadcast_to(off, (E, E)) * mse,
                            axis=-1, keepdims=True).astype(jnp.int32)
    nt_ref[...] = jnp.sum(jnp.broadcast_to(pc / TM, (E, E)) * mse,
                          axis=-1, keepdims=True).astype(jnp.int32)


def _make_router(interpret=False):
    return pl.pallas_call(
        _router_body,
        out_shape=[
            jax.ShapeDtypeStruct((T, 1), jnp.int32),
            jax.ShapeDtypeStruct((T, 128), jnp.float32),
            jax.ShapeDtypeStruct((E, 1), jnp.int32),
            jax.ShapeDtypeStruct((E, 1), jnp.int32),
            jax.ShapeDtypeStruct((E, 1), jnp.int32),
        ],
        interpret=interpret,
    )


def _gmm_body(ea_ref, offs_ref, nt_ref, xs_ref, w1_ref, b1_ref, w2_ref,
              b2_ref, gs_ref, gamma_ref, beta_ref, out_ref):
    s = pl.program_id(0)
    n = nt_ref[s]
    base = pl.multiple_of(offs_ref[s], TM)

    def tile(k, carry):
        r0 = pl.multiple_of(base + k * TM, TM)
        x = xs_ref[pl.ds(r0, TM), :]
        mu = jnp.mean(x, axis=-1, keepdims=True)
        xc = x - mu
        var = jnp.mean(xc * xc, axis=-1, keepdims=True)
        xn = xc / jnp.sqrt(var + 1e-5) * gamma_ref[...] + beta_ref[...]
        h = jnp.dot(xn, w1_ref[0], preferred_element_type=jnp.float32)
        h = jnp.maximum(h + b1_ref[0], 0.0)
        y = jnp.dot(h, w2_ref[0], preferred_element_type=jnp.float32)
        out_ref[pl.ds(r0, TM), :] = \
            (y + b2_ref[0]) * gs_ref[pl.ds(r0, TM), :1]
        return carry

    lax.fori_loop(0, n, tile, 0)


def _make_gmm(interpret=False):
    grid_spec = pltpu.PrefetchScalarGridSpec(
        num_scalar_prefetch=3,
        grid=(E,),
        in_specs=[
            pl.BlockSpec((TPAD, D), lambda s, ea, of, nt: (0, 0)),
            pl.BlockSpec((1, D, F), lambda s, ea, of, nt: (ea[s], 0, 0)),
            pl.BlockSpec((1, 1, F), lambda s, ea, of, nt: (ea[s], 0, 0)),
            pl.BlockSpec((1, F, D), lambda s, ea, of, nt: (ea[s], 0, 0)),
            pl.BlockSpec((1, 1, D), lambda s, ea, of, nt: (ea[s], 0, 0)),
            pl.BlockSpec((TPAD, 128), lambda s, ea, of, nt: (0, 0)),
            pl.BlockSpec((1, D), lambda s, ea, of, nt: (0, 0)),
            pl.BlockSpec((1, D), lambda s, ea, of, nt: (0, 0)),
        ],
        out_specs=pl.BlockSpec((TPAD, D), lambda s, ea, of, nt: (0, 0)),
    )
    return pl.pallas_call(
        _gmm_body,
        grid_spec=grid_spec,
        out_shape=jax.ShapeDtypeStruct((TPAD, D), jnp.float32),
        compiler_params=pltpu.CompilerParams(
            vmem_limit_bytes=100 * 1024 * 1024),
        interpret=interpret,
    )


def _make_dispatch():
    mesh = plsc.VectorSubcoreMesh(core_axis_name="c", subcore_axis_name="s")

    @functools.partial(
        pl.kernel, mesh=mesh,
        out_type=[jax.ShapeDtypeStruct((TPAD, D), jnp.float32),
                  jax.ShapeDtypeStruct((TPAD, 128), jnp.float32)],
        scratch_types=[pltpu.VMEM((TPW,), jnp.int32),
                       pltpu.VMEM((TPW, D), jnp.float32),
                       pltpu.VMEM((TPW, 128), jnp.float32),
                       pltpu.SemaphoreType.DMA,
                       pltpu.SemaphoreType.DMA,
                       pltpu.SemaphoreType.DMA],
    )
    def _dispatch(xn_hbm, dst_hbm, gate_hbm, xs_hbm, gs_hbm,
                  idx_v, rows_v, g_v, sem1, sem2, sem3):
        wid = lax.axis_index("s") * 2 + lax.axis_index("c")
        base = wid * TPW
        ld1 = pltpu.async_copy(dst_hbm.at[pl.ds(base, TPW)], idx_v, sem1)
        ld2 = pltpu.async_copy(xn_hbm.at[pl.ds(base, TPW)], rows_v, sem2)
        ld3 = pltpu.async_copy(gate_hbm.at[pl.ds(base, TPW)], g_v, sem3)
        ld1.wait()
        ld2.wait()
        ld3.wait()
        cp1 = pltpu.async_copy(rows_v, xs_hbm.at[idx_v], sem1)
        cp2 = pltpu.async_copy(g_v, gs_hbm.at[idx_v], sem2)
        cp1.wait()
        cp2.wait()

    return _dispatch


def _make_collect():
    mesh = plsc.VectorSubcoreMesh(core_axis_name="c", subcore_axis_name="s")

    @functools.partial(
        pl.kernel, mesh=mesh,
        out_type=jax.ShapeDtypeStruct((T, D), jnp.float32),
        scratch_types=[pltpu.VMEM((TPW,), jnp.int32),
                       pltpu.VMEM((TPW, D), jnp.float32),
                       pltpu.SemaphoreType.DMA],
    )
    def _collect(ys_hbm, dst_hbm, y_hbm, idx_v, rows_v, sem):
        wid = lax.axis_index("s") * 2 + lax.axis_index("c")
        base = wid * TPW
        pltpu.sync_copy(dst_hbm.at[pl.ds(base, TPW)], idx_v)
        pltpu.async_copy(ys_hbm.at[idx_v], rows_v, sem).wait()
        pltpu.sync_copy(rows_v, y_hbm.at[pl.ds(base, TPW)])

    return _collect


def kernel(x, Wr, br, gamma, beta, W1, b1, W2, b2):
    xf = x.reshape(T, D)
    dst, gate128, ea, offs, nt = _make_router()(xf, Wr, br.reshape(1, E))
    dst1 = dst.reshape(T)
    xs, gs = _make_dispatch()(xf, dst1, gate128)
    ys = _make_gmm()(ea.reshape(E), offs.reshape(E), nt.reshape(E),
                     xs, W1, b1.reshape(E, 1, F),
                     W2, b2.reshape(E, 1, D), gs,
                     gamma.reshape(1, D), beta.reshape(1, D))
    y = _make_collect()(ys, dst1)
    return y.reshape(1, T, D)


# speedup vs baseline: 2.0748x; 1.0001x over previous
import functools

import jax
import jax.numpy as jnp
from jax import lax
from jax.experimental import pallas as pl
from jax.experimental.pallas import tpu as pltpu
from jax.experimental.pallas import tpu_sc as plsc

E = 8
D = 768
F = 3072
T = 2048
TM = 128
TPAD = T + E * TM
RC = 256

NW = 32
TPW = T // NW


def _router_body(x_ref, wr_ref, br_ref, dst_ref, gate_ref,
                 ea_ref, offs_ref, nt_ref):
    x = x_ref[...]
    logits = jnp.dot(x, wr_ref[...], preferred_element_type=jnp.float32)
    logits = logits + br_ref[...]
    m = jnp.max(logits, axis=-1, keepdims=True)
    ex = jnp.exp(logits - m)
    den = jnp.sum(ex, axis=-1, keepdims=True)
    gate = jnp.max(ex, axis=-1, keepdims=True) / den
    eidx = lax.broadcasted_iota(jnp.int32, (T, E), 1)
    idx = jnp.min(jnp.where(logits >= m, eidx, E), axis=-1, keepdims=True)
    onehot = (eidx == idx).astype(jnp.float32)

    ci = lax.broadcasted_iota(jnp.int32, (RC, RC), 0)
    cj = lax.broadcasted_iota(jnp.int32, (RC, RC), 1)
    ltf = (cj < ci).astype(jnp.float32)
    carry = jnp.zeros((1, E), jnp.float32)
    chunks = []
    for c in range(T // RC):
        oh_c = lax.slice(onehot, (c * RC, 0), ((c + 1) * RC, E))
        r_c = jnp.dot(ltf, oh_c, preferred_element_type=jnp.float32)
        chunks.append(r_c + carry)
        carry = carry + jnp.sum(oh_c, axis=0, keepdims=True)
    ranks = jnp.concatenate(chunks, axis=0)
    counts = carry

    pc = jnp.ceil(counts / TM) * TM
    ei = lax.broadcasted_iota(jnp.int32, (E, E), 0)
    ej = lax.broadcasted_iota(jnp.int32, (E, E), 1)
    le = (ei <= ej).astype(jnp.float32)
    cum = jnp.dot(pc, le, preferred_element_type=jnp.float32)
    off = cum - pc

    rank_t = jnp.sum(ranks * onehot, axis=-1, keepdims=True)
    off_t = jnp.sum(off * onehot, axis=-1, keepdims=True)
    dst_ref[...] = (rank_t + off_t).astype(jnp.int32)
    gate_ref[...] = jnp.broadcast_to(gate, (T, 128))

    present = (counts > 0.0).astype(jnp.float32)
    lt_strict = (ei < ej).astype(jnp.float32)
    rankp = jnp.dot(present, lt_strict,
                    preferred_element_type=jnp.float32)
    np_ = jnp.sum(present)
    srow = lax.broadcasted_iota(jnp.int32, (E, E), 0).astype(jnp.float32)
    ecol = lax.broadcasted_iota(jnp.int32, (E, E), 1).astype(jnp.float32)
    mse = (jnp.broadcast_to(rankp, (E, E)) == srow) * \
        jnp.broadcast_to(present, (E, E))
    ea_col = jnp.sum(ecol * mse, axis=-1, keepdims=True)
    is_last = present * (rankp == (np_ - 1.0)).astype(jnp.float32)
    lastp = jnp.sum(is_last * ecol[:1, :])
    beyond = (lax.broadcasted_iota(jnp.int32, (E, 1), 0).astype(jnp.float32)
              >= np_).astype(jnp.float32)
    ea_ref[...] = (ea_col + beyond * lastp).astype(jnp.int32)
    offs_ref[...] = jnp.sum(jnp.broadcast_to(off, (E, E)) * mse,
                            axis=-1, keepdims=True).astype(jnp.int32)
    nt_ref[...] = jnp.sum(jnp.broadcast_to(pc / TM, (E, E)) * mse,
                          axis=-1, keepdims=True).astype(jnp.int32)


def _make_router(interpret=False):
    return pl.pallas_call(
        _router_body,
        out_shape=[
            jax.ShapeDtypeStruct((T, 1), jnp.int32),
            jax.ShapeDtypeStruct((T, 128), jnp.float32),
            jax.ShapeDtypeStruct((E, 1), jnp.int32),
            jax.ShapeDtypeStruct((E, 1), jnp.int32),
            jax.ShapeDtypeStruct((E, 1), jnp.int32),
        ],
        interpret=interpret,
    )


def _gmm_body(ea_ref, offs_ref, nt_ref, xs_ref, w1_ref, b1_ref, w2_ref,
              b2_ref, gs_ref, gamma_ref, beta_ref, out_ref):
    s = pl.program_id(0)
    n = nt_ref[s]
    base = pl.multiple_of(offs_ref[s], TM)

    def tile(k, carry):
        r0 = pl.multiple_of(base + k * TM, TM)
        x = xs_ref[pl.ds(r0, TM), :]
        mu = jnp.mean(x, axis=-1, keepdims=True)
        xc = x - mu
        var = jnp.mean(xc * xc, axis=-1, keepdims=True)
        xn = xc / jnp.sqrt(var + 1e-5) * gamma_ref[...] + beta_ref[...]
        h = jnp.dot(xn, w1_ref[0], preferred_element_type=jnp.float32)
        h = jnp.maximum(h + b1_ref[0], 0.0)
        y = jnp.dot(h, w2_ref[0], preferred_element_type=jnp.float32)
        out_ref[pl.ds(r0, TM), :] = \
            (y + b2_ref[0]) * gs_ref[pl.ds(r0, TM), :1]
        return carry

    lax.fori_loop(0, n, tile, 0)


def _make_gmm(interpret=False):
    grid_spec = pltpu.PrefetchScalarGridSpec(
        num_scalar_prefetch=3,
        grid=(E,),
        in_specs=[
            pl.BlockSpec((TPAD, D), lambda s, ea, of, nt: (0, 0)),
            pl.BlockSpec((1, D, F), lambda s, ea, of, nt: (ea[s], 0, 0)),
            pl.BlockSpec((1, 1, F), lambda s, ea, of, nt: (ea[s], 0, 0)),
            pl.BlockSpec((1, F, D), lambda s, ea, of, nt: (ea[s], 0, 0)),
            pl.BlockSpec((1, 1, D), lambda s, ea, of, nt: (ea[s], 0, 0)),
            pl.BlockSpec((TPAD, 128), lambda s, ea, of, nt: (0, 0)),
            pl.BlockSpec((1, D), lambda s, ea, of, nt: (0, 0)),
            pl.BlockSpec((1, D), lambda s, ea, of, nt: (0, 0)),
        ],
        out_specs=pl.BlockSpec((TPAD, D), lambda s, ea, of, nt: (0, 0)),
    )
    return pl.pallas_call(
        _gmm_body,
        grid_spec=grid_spec,
        out_shape=jax.ShapeDtypeStruct((TPAD, D), jnp.float32),
        compiler_params=pltpu.CompilerParams(
            vmem_limit_bytes=100 * 1024 * 1024),
        interpret=interpret,
    )


def _make_dispatch():
    mesh = plsc.VectorSubcoreMesh(core_axis_name="c", subcore_axis_name="s")

    @functools.partial(
        pl.kernel, mesh=mesh,
        out_type=[jax.ShapeDtypeStruct((TPAD, D), jnp.float32),
                  jax.ShapeDtypeStruct((TPAD, 128), jnp.float32)],
        scratch_types=[pltpu.VMEM((TPW,), jnp.int32),
                       pltpu.VMEM((TPW, D), jnp.float32),
                       pltpu.VMEM((TPW, 128), jnp.float32),
                       pltpu.SemaphoreType.DMA,
                       pltpu.SemaphoreType.DMA,
                       pltpu.SemaphoreType.DMA],
    )
    def _dispatch(xn_hbm, dst_hbm, gate_hbm, xs_hbm, gs_hbm,
                  idx_v, rows_v, g_v, sem1, sem2, sem3):
        wid = lax.axis_index("s") * 2 + lax.axis_index("c")
        base = wid * TPW
        ld1 = pltpu.async_copy(dst_hbm.at[pl.ds(base, TPW)], idx_v, sem1)
        ld2 = pltpu.async_copy(xn_hbm.at[pl.ds(base, TPW)], rows_v, sem2)
        ld3 = pltpu.async_copy(gate_hbm.at[pl.ds(base, TPW)], g_v, sem3)
        ld1.wait()
        ld2.wait()
        ld3.wait()
        cp1 = pltpu.async_copy(rows_v, xs_hbm.at[idx_v], sem1)
        cp2 = pltpu.async_copy(g_v, gs_hbm.at[idx_v], sem2)
        cp1.wait()
        cp2.wait()

    return _dispatch


def _make_collect():
    mesh = plsc.VectorSubcoreMesh(core_axis_name="c", subcore_axis_name="s")

    @functools.partial(
        pl.kernel, mesh=mesh,
        out_type=jax.ShapeDtypeStruct((T, D), jnp.float32),
        scratch_types=[pltpu.VMEM((TPW,), jnp.int32),
                       pltpu.VMEM((TPW, D), jnp.float32),
                       pltpu.SemaphoreType.DMA],
    )
    def _collect(ys_hbm, dst_hbm, y_hbm, idx_v, rows_v, sem):
        wid = lax.axis_index("s") * 2 + lax.axis_index("c")
        base = wid * TPW
        pltpu.sync_copy(dst_hbm.at[pl.ds(base, TPW)], idx_v)
        pltpu.async_copy(ys_hbm.at[idx_v], rows_v, sem).wait()
        pltpu.sync_copy(rows_v, y_hbm.at[pl.ds(base, TPW)])

    return _collect


def kernel(x, Wr, br, gamma, beta, W1, b1, W2, b2):
    xf = x.reshape(T, D)
    dst, gate128, ea, offs, nt = _make_router()(xf, Wr, br.reshape(1, E))
    dst1 = dst.reshape(T)
    xs, gs = _make_dispatch()(xf, dst1, gate128)
    ys = _make_gmm()(ea.reshape(E), offs.reshape(E), nt.reshape(E),
                     xs, W1, b1.reshape(E, 1, F),
                     W2, b2.reshape(E, 1, D), gs,
                     gamma.reshape(1, D), beta.reshape(1, D))
    y = _make_collect()(ys, dst1)
    return y.reshape(1, T, D)
